# Initial kernel scaffold; baseline (speedup 1.0000x reference)
import functools

import jax
import jax.numpy as jnp
from jax import lax
from jax.experimental import pallas as pl
from jax.experimental.pallas import tpu as pltpu
from jax.experimental.pallas import tpu_sc as plsc

N = 10000
E = 320000
DIN = 128
DH = 128
DOUT = 2
G = 64

NC = 2
NS = 16
NW = NC * NS
EPW = E // NW
KC = 125
NCH = EPW // KC
RPT = N // NS
ZR = 125
DP = 144
DQ = 16

BN = 400
NB = N // BN


def _make_seg_sum(D):
    mesh = plsc.VectorSubcoreMesh(core_axis_name="core", subcore_axis_name="subcore")

    def body(vals_hbm, src_hbm, dst_hbm, zeros_hbm, out_hbm,
             src_v, dst_v, rows_v, zbuf_v, acc_sh):
        c = lax.axis_index("core")
        s = lax.axis_index("subcore")
        wid = c * NS + s
        pltpu.sync_copy(src_hbm.at[wid], src_v)
        pltpu.sync_copy(dst_hbm.at[wid], dst_v)
        pltpu.sync_copy(zeros_hbm, zbuf_v)
        for t in range(RPT // ZR):
            pltpu.sync_copy(zbuf_v, acc_sh.at[pl.ds(s * RPT + t * ZR, ZR)])
        plsc.subcore_barrier()

        @pl.loop(0, NCH)
        def _(j):
            pltpu.sync_copy(vals_hbm.at[src_v.at[j]], rows_v)
            pltpu.sync_copy(rows_v, acc_sh.at[dst_v.at[j]], add=True)

        plsc.subcore_barrier()
        pltpu.sync_copy(acc_sh.at[pl.ds(s * RPT, RPT)],
                        out_hbm.at[c, pl.ds(s * RPT, RPT)])

    return pl.kernel(
        body,
        out_type=jax.ShapeDtypeStruct((NC, N, D), jnp.float32),
        mesh=mesh,
        scratch_types=[
            pltpu.VMEM((NCH, KC), jnp.int32),
            pltpu.VMEM((NCH, KC), jnp.int32),
            pltpu.VMEM((KC, D), jnp.float32),
            pltpu.VMEM((ZR, D), jnp.float32),
            pltpu.VMEM_SHARED((N, D), jnp.float32),
        ],
    )


_seg_sum_dp = _make_seg_sum(DP)
_seg_sum_dq = _make_seg_sum(DQ)


def _onehot(bf_blk):
    gi = lax.broadcasted_iota(jnp.float32, (G, BN), 0)
    b = jnp.broadcast_to(bf_blk, (G, BN))
    return jnp.where(gi == b, 1.0, 0.0).astype(jnp.float32)


def _tc1_body(agg_ref, x_ref, bf_ref, wl1_ref, bl1_ref, wr1_ref, wl2p_ref,
              p_ref, cinv_ref, ph1_ref):
    i = pl.program_id(0)
    a = agg_ref[0] + agg_ref[1]
    feats = a[:, :DIN]
    cnt = a[:, DIN:DIN + 1]
    cinv = 1.0 / jnp.maximum(cnt, 1.0)
    mean = feats * cinv
    h1 = jnp.dot(mean, wl1_ref[...], preferred_element_type=jnp.float32)
    h1 = h1 + bl1_ref[...]
    h1 = h1 + jnp.dot(x_ref[...], wr1_ref[...], preferred_element_type=jnp.float32)
    h1 = jnp.maximum(h1, 0.0)
    p_ref[...] = jnp.dot(h1, wl2p_ref[...], preferred_element_type=jnp.float32)
    cinv_ref[...] = jnp.broadcast_to(cinv, (BN, DQ))
    oh = _onehot(bf_ref[...])

    @pl.when(i == 0)
    def _():
        ph1_ref[...] = jnp.zeros_like(ph1_ref)

    ph1_ref[...] += jnp.dot(oh, h1, preferred_element_type=jnp.float32)


def _tc1(agg, x, bf, wl1, bl1, wr1, wl2p):
    return pl.pallas_call(
        _tc1_body,
        grid=(NB,),
        in_specs=[
            pl.BlockSpec((NC, BN, DP), lambda i: (0, i, 0)),
            pl.BlockSpec((BN, DIN), lambda i: (i, 0)),
            pl.BlockSpec((1, BN), lambda i: (0, i)),
            pl.BlockSpec((DIN, DH), lambda i: (0, 0)),
            pl.BlockSpec((1, DH), lambda i: (0, 0)),
            pl.BlockSpec((DIN, DH), lambda i: (0, 0)),
            pl.BlockSpec((DH, DQ), lambda i: (0, 0)),
        ],
        out_specs=[
            pl.BlockSpec((BN, DQ), lambda i: (i, 0)),
            pl.BlockSpec((BN, DQ), lambda i: (i, 0)),
            pl.BlockSpec((G, DH), lambda i: (0, 0)),
        ],
        out_shape=[
            jax.ShapeDtypeStruct((N, DQ), jnp.float32),
            jax.ShapeDtypeStruct((N, DQ), jnp.float32),
            jax.ShapeDtypeStruct((G, DH), jnp.float32),
        ],
    )(agg, x, bf, wl1, bl1, wr1, wl2p)


def _tc2_body(q_ref, cinv_ref, bf_ref, ph1_ref, wr2p_ref, bl2p_ref,
              hg_ref, lsm_ref, pq_acc, gc_acc):
    i = pl.program_id(0)

    @pl.when(i == 0)
    def _():
        pq_acc[...] = jnp.zeros_like(pq_acc)
        gc_acc[...] = jnp.zeros_like(gc_acc)

    qq = (q_ref[0] + q_ref[1]) * cinv_ref[...]
    oh = _onehot(bf_ref[...])
    pq_acc[...] += jnp.dot(oh, qq, preferred_element_type=jnp.float32)
    gc_acc[...] += jnp.dot(oh, jnp.ones((BN, DQ), jnp.float32),
                           preferred_element_type=jnp.float32)

    @pl.when(i == NB - 1)
    def _():
        gc = gc_acc[:, 0:1]
        num = (pq_acc[...]
               + jnp.dot(ph1_ref[...], wr2p_ref[...],
                         preferred_element_type=jnp.float32)
               + gc * bl2p_ref[...])
        hg = num / jnp.maximum(gc, 1.0)
        lanes = lax.broadcasted_iota(jnp.int32, (G, DQ), 1)
        msk = lanes < DOUT
        mx = jnp.max(jnp.where(msk, hg, -1e30), axis=1, keepdims=True)
        sh = hg - mx
        e = jnp.where(msk, jnp.exp(sh), 0.0)
        se = jnp.sum(e, axis=1, keepdims=True)
        hg_ref[...] = hg
        lsm_ref[...] = sh - jnp.log(se)


def _tc2(q, cinv, bf, ph1, wr2p, bl2p):
    return pl.pallas_call(
        _tc2_body,
        grid=(NB,),
        in_specs=[
            pl.BlockSpec((NC, BN, DQ), lambda i: (0, i, 0)),
            pl.BlockSpec((BN, DQ), lambda i: (i, 0)),
            pl.BlockSpec((1, BN), lambda i: (0, i)),
            pl.BlockSpec((G, DH), lambda i: (0, 0)),
            pl.BlockSpec((DH, DQ), lambda i: (0, 0)),
            pl.BlockSpec((1, DQ), lambda i: (0, 0)),
        ],
        out_specs=[
            pl.BlockSpec((G, DQ), lambda i: (0, 0)),
            pl.BlockSpec((G, DQ), lambda i: (0, 0)),
        ],
        out_shape=[
            jax.ShapeDtypeStruct((G, DQ), jnp.float32),
            jax.ShapeDtypeStruct((G, DQ), jnp.float32),
        ],
        scratch_shapes=[
            pltpu.VMEM((G, DQ), jnp.float32),
            pltpu.VMEM((G, DQ), jnp.float32),
        ],
    )(q, cinv, bf, ph1, wr2p, bl2p)


def kernel(x, edge_index, batch, W_l1, b_l1, W_r1, W_l2, b_l2, W_r2):
    f32 = jnp.float32
    x_aug = jnp.concatenate(
        [x, jnp.ones((N, 1), f32), jnp.zeros((N, DP - DIN - 1), f32)], axis=1)
    src = edge_index[0].reshape(NW, NCH, KC)
    dst = edge_index[1].reshape(NW, NCH, KC)

    part1 = _seg_sum_dp(x_aug, src, dst, jnp.zeros((ZR, DP), f32))

    bf = batch.astype(f32).reshape(1, N)
    wl2p = jnp.pad(W_l2, ((0, 0), (0, DQ - DOUT)))
    p, cinv16, ph1 = _tc1(part1, x, bf, W_l1, b_l1.reshape(1, DH), W_r1, wl2p)

    part2 = _seg_sum_dq(p, src, dst, jnp.zeros((ZR, DQ), f32))

    wr2p = jnp.pad(W_r2, ((0, 0), (0, DQ - DOUT)))
    bl2p = jnp.pad(b_l2, (0, DQ - DOUT)).reshape(1, DQ)
    hg, lsm = _tc2(part2, cinv16, bf, ph1, wr2p, bl2p)
    return hg[:, :DOUT], lsm[:, :DOUT]


# speedup vs baseline: 8.4464x; 8.4464x over previous
import functools

import jax
import jax.numpy as jnp
from jax import lax
from jax.experimental import pallas as pl
from jax.experimental.pallas import tpu as pltpu
from jax.experimental.pallas import tpu_sc as plsc

N = 10000
E = 320000
DIN = 128
DH = 128
DOUT = 2
G = 64

NC = 2
NS = 16
NW = NC * NS
EPW = E // NW
KC = 125
NCH = EPW // KC
RPT = N // NS
ZR = 125
DA = 80
DB = 64
DQ = 16

BN = 400
NB = N // BN


def _make_seg_sum(D):
    mesh = plsc.VectorSubcoreMesh(core_axis_name="core", subcore_axis_name="subcore")

    def body(vals_hbm, src_hbm, dst_hbm, zeros_hbm, out_hbm,
             src_v, dst_v, rows_v, zbuf_v, acc_sh):
        c = lax.axis_index("core")
        s = lax.axis_index("subcore")
        wid = c * NS + s
        pltpu.sync_copy(src_hbm.at[wid], src_v)
        pltpu.sync_copy(dst_hbm.at[wid], dst_v)
        pltpu.sync_copy(zeros_hbm, zbuf_v)
        for t in range(RPT // ZR):
            pltpu.sync_copy(zbuf_v, acc_sh.at[pl.ds(s * RPT + t * ZR, ZR)])
        plsc.subcore_barrier()

        @pl.loop(0, NCH)
        def _(j):
            pltpu.sync_copy(vals_hbm.at[src_v.at[j]], rows_v)
            pltpu.sync_copy(rows_v, acc_sh.at[dst_v.at[j]], add=True)

        plsc.subcore_barrier()
        pltpu.sync_copy(acc_sh.at[pl.ds(s * RPT, RPT)],
                        out_hbm.at[c, pl.ds(s * RPT, RPT)])

    return pl.kernel(
        body,
        out_type=jax.ShapeDtypeStruct((NC, N, D), jnp.float32),
        mesh=mesh,
        compiler_params=pltpu.CompilerParams(use_tc_tiling_on_sc=False),
        scratch_types=[
            pltpu.VMEM((NCH, KC), jnp.int32),
            pltpu.VMEM((NCH, KC), jnp.int32),
            pltpu.VMEM((KC, D), jnp.float32),
            pltpu.VMEM((ZR, D), jnp.float32),
            pltpu.VMEM_SHARED((N, D), jnp.float32),
        ],
    )


_make_seg_sum = functools.cache(_make_seg_sum)


def _onehot(bf_ref):
    gi = lax.broadcasted_iota(jnp.int32, (G, BN), 0).astype(jnp.float32)
    b = jnp.broadcast_to(bf_ref[0], (G, BN))
    return jnp.where(gi == b, 1.0, 0.0).astype(jnp.float32)


def _tc1_body(agga_ref, aggb_ref, x_ref, bf_ref, wl1_ref, bl1_ref, wr1_ref,
              wl2p_ref, p_ref, cinv_ref, ph1_ref):
    i = pl.program_id(0)
    aa = agga_ref[0] + agga_ref[1]
    ab = aggb_ref[0] + aggb_ref[1]
    feats = jnp.concatenate([aa, ab[:, :DIN - DA]], axis=1)
    cnt = ab[:, DIN - DA:DIN - DA + 1]
    cinv = 1.0 / jnp.maximum(cnt, 1.0)
    mean = feats * cinv
    h1 = jnp.dot(mean, wl1_ref[...], preferred_element_type=jnp.float32)
    h1 = h1 + bl1_ref[...]
    h1 = h1 + jnp.dot(x_ref[...], wr1_ref[...], preferred_element_type=jnp.float32)
    h1 = jnp.maximum(h1, 0.0)
    p_ref[...] = jnp.dot(h1, wl2p_ref[...], preferred_element_type=jnp.float32)
    cinv_ref[...] = jnp.broadcast_to(cinv, (BN, DQ))
    oh = _onehot(bf_ref)

    @pl.when(i == 0)
    def _():
        ph1_ref[...] = jnp.zeros_like(ph1_ref)

    ph1_ref[...] += jnp.dot(oh, h1, preferred_element_type=jnp.float32)


def _tc1(agga, aggb, x, bf, wl1, bl1, wr1, wl2p):
    return pl.pallas_call(
        _tc1_body,
        grid=(NB,),
        in_specs=[
            pl.BlockSpec((NC, BN, DA), lambda i: (0, i, 0)),
            pl.BlockSpec((NC, BN, DB), lambda i: (0, i, 0)),
            pl.BlockSpec((BN, DIN), lambda i: (i, 0)),
            pl.BlockSpec((1, 1, BN), lambda i: (i, 0, 0)),
            pl.BlockSpec((DIN, DH), lambda i: (0, 0)),
            pl.BlockSpec((1, DH), lambda i: (0, 0)),
            pl.BlockSpec((DIN, DH), lambda i: (0, 0)),
            pl.BlockSpec((DH, DQ), lambda i: (0, 0)),
        ],
        out_specs=[
            pl.BlockSpec((BN, DQ), lambda i: (i, 0)),
            pl.BlockSpec((BN, DQ), lambda i: (i, 0)),
            pl.BlockSpec((G, DH), lambda i: (0, 0)),
        ],
        out_shape=[
            jax.ShapeDtypeStruct((N, DQ), jnp.float32),
            jax.ShapeDtypeStruct((N, DQ), jnp.float32),
            jax.ShapeDtypeStruct((G, DH), jnp.float32),
        ],
    )(agga, aggb, x, bf, wl1, bl1, wr1, wl2p)


def _tc2_body(q_ref, cinv_ref, bf_ref, ph1_ref, wr2p_ref, bl2p_ref,
              hg_ref, lsm_ref, pq_acc, gc_acc):
    i = pl.program_id(0)

    @pl.when(i == 0)
    def _():
        pq_acc[...] = jnp.zeros_like(pq_acc)
        gc_acc[...] = jnp.zeros_like(gc_acc)

    qq = (q_ref[0] + q_ref[1]) * cinv_ref[...]
    oh = _onehot(bf_ref)
    pq_acc[...] += jnp.dot(oh, qq, preferred_element_type=jnp.float32)
    gc_acc[...] += jnp.dot(oh, jnp.ones((BN, DQ), jnp.float32),
                           preferred_element_type=jnp.float32)

    @pl.when(i == NB - 1)
    def _():
        gc = gc_acc[:, 0:1]
        num = (pq_acc[...]
               + jnp.dot(ph1_ref[...], wr2p_ref[...],
                         preferred_element_type=jnp.float32)
               + gc * bl2p_ref[...])
        hg = num / jnp.maximum(gc, 1.0)
        lanes = lax.broadcasted_iota(jnp.int32, (G, DQ), 1)
        msk = lanes < DOUT
        mx = jnp.max(jnp.where(msk, hg, -1e30), axis=1, keepdims=True)
        sh = hg - mx
        e = jnp.where(msk, jnp.exp(sh), 0.0)
        se = jnp.sum(e, axis=1, keepdims=True)
        hg_ref[...] = hg
        lsm_ref[...] = sh - jnp.log(se)


def _tc2(q, cinv, bf, ph1, wr2p, bl2p):
    return pl.pallas_call(
        _tc2_body,
        grid=(NB,),
        in_specs=[
            pl.BlockSpec((NC, BN, DQ), lambda i: (0, i, 0)),
            pl.BlockSpec((BN, DQ), lambda i: (i, 0)),
            pl.BlockSpec((1, 1, BN), lambda i: (i, 0, 0)),
            pl.BlockSpec((G, DH), lambda i: (0, 0)),
            pl.BlockSpec((DH, DQ), lambda i: (0, 0)),
            pl.BlockSpec((1, DQ), lambda i: (0, 0)),
        ],
        out_specs=[
            pl.BlockSpec((G, DQ), lambda i: (0, 0)),
            pl.BlockSpec((G, DQ), lambda i: (0, 0)),
        ],
        out_shape=[
            jax.ShapeDtypeStruct((G, DQ), jnp.float32),
            jax.ShapeDtypeStruct((G, DQ), jnp.float32),
        ],
        scratch_shapes=[
            pltpu.VMEM((G, DQ), jnp.float32),
            pltpu.VMEM((G, DQ), jnp.float32),
        ],
    )(q, cinv, bf, ph1, wr2p, bl2p)


def kernel(x, edge_index, batch, W_l1, b_l1, W_r1, W_l2, b_l2, W_r2):
    f32 = jnp.float32
    xa = x[:, :DA]
    xb = jnp.concatenate(
        [x[:, DA:DIN], jnp.ones((N, 1), f32),
         jnp.zeros((N, DB - (DIN - DA) - 1), f32)], axis=1)
    src = edge_index[0].reshape(NW, NCH, KC)
    dst = edge_index[1].reshape(NW, NCH, KC)

    part1a = _make_seg_sum(DA)(xa, src, dst, jnp.zeros((ZR, DA), f32))
    part1b = _make_seg_sum(DB)(xb, src, dst, jnp.zeros((ZR, DB), f32))

    bf = batch.astype(f32).reshape(NB, 1, BN)
    wl2p = jnp.pad(W_l2, ((0, 0), (0, DQ - DOUT)))
    p, cinv16, ph1 = _tc1(part1a, part1b, x, bf, W_l1, b_l1.reshape(1, DH),
                          W_r1, wl2p)

    part2 = _make_seg_sum(DQ)(p, src, dst, jnp.zeros((ZR, DQ), f32))

    wr2p = jnp.pad(W_r2, ((0, 0), (0, DQ - DOUT)))
    bl2p = jnp.pad(b_l2, (0, DQ - DOUT)).reshape(1, DQ)
    hg, lsm = _tc2(part2, cinv16, bf, ph1, wr2p, bl2p)
    return hg[:, :DOUT], lsm[:, :DOUT]


# speedup vs baseline: 9.4409x; 1.1177x over previous
import functools

import jax
import jax.numpy as jnp
from jax import lax
from jax.experimental import pallas as pl
from jax.experimental.pallas import tpu as pltpu
from jax.experimental.pallas import tpu_sc as plsc

N = 10000
E = 320000
DIN = 128
DH = 128
DOUT = 2
G = 64

NC = 2
NS = 16
NW = NC * NS
EPW = E // NW
KC = 125
NCH = EPW // KC
RPT = N // NS
ZR = 125
DC = 144
DQ = 16

BN = 400
NB = N // BN


def _make_seg_sum(D):
    mesh = plsc.VectorSubcoreMesh(core_axis_name="core", subcore_axis_name="subcore")

    def body(vals_hbm, src_hbm, dst_hbm, zeros_hbm, out_hbm,
             src_v, dst_v, rows_v, acc_sh):
        c = lax.axis_index("core")
        s = lax.axis_index("subcore")
        wid = c * NS + s
        pltpu.sync_copy(src_hbm.at[wid], src_v)
        pltpu.sync_copy(dst_hbm.at[wid], dst_v)
        pltpu.sync_copy(zeros_hbm, rows_v)
        for t in range(RPT // ZR):
            pltpu.sync_copy(rows_v, acc_sh.at[pl.ds(s * RPT + t * ZR, ZR)])
        plsc.subcore_barrier()

        @pl.loop(0, NCH)
        def _(j):
            pltpu.sync_copy(vals_hbm.at[src_v.at[j]], rows_v)
            pltpu.sync_copy(rows_v, acc_sh.at[dst_v.at[j]], add=True)

        plsc.subcore_barrier()
        pltpu.sync_copy(acc_sh.at[pl.ds(s * RPT, RPT)],
                        out_hbm.at[c, pl.ds(s * RPT, RPT)])

    return pl.kernel(
        body,
        out_type=jax.ShapeDtypeStruct((NC, N, D), jnp.float32),
        mesh=mesh,
        compiler_params=pltpu.CompilerParams(use_tc_tiling_on_sc=False),
        scratch_types=[
            pltpu.VMEM((NCH, KC), jnp.int32),
            pltpu.VMEM((NCH, KC), jnp.int32),
            pltpu.VMEM((KC, D), jnp.float32),
            pltpu.VMEM_SHARED((N, D), jnp.float32),
        ],
    )


_make_seg_sum = functools.cache(_make_seg_sum)


def _onehot(bf_ref):
    gi = lax.broadcasted_iota(jnp.int32, (G, BN), 0).astype(jnp.float32)
    b = jnp.broadcast_to(bf_ref[0], (G, BN))
    return jnp.where(gi == b, 1.0, 0.0).astype(jnp.float32)


def _tc1_body(agg_ref, x_ref, bf_ref, wl1_ref, bl1_ref, wr1_ref,
              wl2p_ref, p_ref, cinv_ref, ph1_ref):
    i = pl.program_id(0)
    aa = agg_ref[0] + agg_ref[1]
    feats = aa[:, :DIN]
    cnt = aa[:, DIN:DIN + 1]
    cinv = 1.0 / jnp.maximum(cnt, 1.0)
    mean = feats * cinv
    h1 = jnp.dot(mean, wl1_ref[...], preferred_element_type=jnp.float32)
    h1 = h1 + bl1_ref[...]
    h1 = h1 + jnp.dot(x_ref[...], wr1_ref[...], preferred_element_type=jnp.float32)
    h1 = jnp.maximum(h1, 0.0)
    p_ref[...] = jnp.dot(h1, wl2p_ref[...], preferred_element_type=jnp.float32)
    cinv_ref[...] = jnp.broadcast_to(cinv, (BN, DQ))
    oh = _onehot(bf_ref)

    @pl.when(i == 0)
    def _():
        ph1_ref[...] = jnp.zeros_like(ph1_ref)

    ph1_ref[...] += jnp.dot(oh, h1, preferred_element_type=jnp.float32)


def _tc1(agg, x, bf, wl1, bl1, wr1, wl2p):
    return pl.pallas_call(
        _tc1_body,
        grid=(NB,),
        in_specs=[
            pl.BlockSpec((NC, BN, DC), lambda i: (0, i, 0)),
            pl.BlockSpec((BN, DIN), lambda i: (i, 0)),
            pl.BlockSpec((1, 1, BN), lambda i: (i, 0, 0)),
            pl.BlockSpec((DIN, DH), lambda i: (0, 0)),
            pl.BlockSpec((1, DH), lambda i: (0, 0)),
            pl.BlockSpec((DIN, DH), lambda i: (0, 0)),
            pl.BlockSpec((DH, DQ), lambda i: (0, 0)),
        ],
        out_specs=[
            pl.BlockSpec((BN, DQ), lambda i: (i, 0)),
            pl.BlockSpec((BN, DQ), lambda i: (i, 0)),
            pl.BlockSpec((G, DH), lambda i: (0, 0)),
        ],
        out_shape=[
            jax.ShapeDtypeStruct((N, DQ), jnp.float32),
            jax.ShapeDtypeStruct((N, DQ), jnp.float32),
            jax.ShapeDtypeStruct((G, DH), jnp.float32),
        ],
    )(agg, x, bf, wl1, bl1, wr1, wl2p)


def _tc2_body(q_ref, cinv_ref, bf_ref, ph1_ref, wr2p_ref, bl2p_ref,
              hg_ref, lsm_ref, pq_acc, gc_acc):
    i = pl.program_id(0)

    @pl.when(i == 0)
    def _():
        pq_acc[...] = jnp.zeros_like(pq_acc)
        gc_acc[...] = jnp.zeros_like(gc_acc)

    qq = (q_ref[0] + q_ref[1]) * cinv_ref[...]
    oh = _onehot(bf_ref)
    pq_acc[...] += jnp.dot(oh, qq, preferred_element_type=jnp.float32)
    gc_acc[...] += jnp.dot(oh, jnp.ones((BN, DQ), jnp.float32),
                           preferred_element_type=jnp.float32)

    @pl.when(i == NB - 1)
    def _():
        gc = gc_acc[:, 0:1]
        num = (pq_acc[...]
               + jnp.dot(ph1_ref[...], wr2p_ref[...],
                         preferred_element_type=jnp.float32)
               + gc * bl2p_ref[...])
        hg = num / jnp.maximum(gc, 1.0)
        lanes = lax.broadcasted_iota(jnp.int32, (G, DQ), 1)
        msk = lanes < DOUT
        mx = jnp.max(jnp.where(msk, hg, -1e30), axis=1, keepdims=True)
        sh = hg - mx
        e = jnp.where(msk, jnp.exp(sh), 0.0)
        se = jnp.sum(e, axis=1, keepdims=True)
        hg_ref[...] = hg
        lsm_ref[...] = sh - jnp.log(se)


def _tc2(q, cinv, bf, ph1, wr2p, bl2p):
    return pl.pallas_call(
        _tc2_body,
        grid=(NB,),
        in_specs=[
            pl.BlockSpec((NC, BN, DQ), lambda i: (0, i, 0)),
            pl.BlockSpec((BN, DQ), lambda i: (i, 0)),
            pl.BlockSpec((1, 1, BN), lambda i: (i, 0, 0)),
            pl.BlockSpec((G, DH), lambda i: (0, 0)),
            pl.BlockSpec((DH, DQ), lambda i: (0, 0)),
            pl.BlockSpec((1, DQ), lambda i: (0, 0)),
        ],
        out_specs=[
            pl.BlockSpec((G, DQ), lambda i: (0, 0)),
            pl.BlockSpec((G, DQ), lambda i: (0, 0)),
        ],
        out_shape=[
            jax.ShapeDtypeStruct((G, DQ), jnp.float32),
            jax.ShapeDtypeStruct((G, DQ), jnp.float32),
        ],
        scratch_shapes=[
            pltpu.VMEM((G, DQ), jnp.float32),
            pltpu.VMEM((G, DQ), jnp.float32),
        ],
    )(q, cinv, bf, ph1, wr2p, bl2p)


def kernel(x, edge_index, batch, W_l1, b_l1, W_r1, W_l2, b_l2, W_r2):
    f32 = jnp.float32
    xc = jnp.concatenate(
        [x, jnp.ones((N, 1), f32), jnp.zeros((N, DC - DIN - 1), f32)], axis=1)
    src = edge_index[0].reshape(NW, NCH, KC)
    dst = edge_index[1].reshape(NW, NCH, KC)

    part1 = _make_seg_sum(DC)(xc, src, dst, jnp.zeros((ZR, DC), f32))

    bf = batch.astype(f32).reshape(NB, 1, BN)
    wl2p = jnp.pad(W_l2, ((0, 0), (0, DQ - DOUT)))
    p, cinv16, ph1 = _tc1(part1, x, bf, W_l1, b_l1.reshape(1, DH),
                          W_r1, wl2p)

    part2 = _make_seg_sum(DQ)(p, src, dst, jnp.zeros((ZR, DQ), f32))

    wr2p = jnp.pad(W_r2, ((0, 0), (0, DQ - DOUT)))
    bl2p = jnp.pad(b_l2, (0, DQ - DOUT)).reshape(1, DQ)
    hg, lsm = _tc2(part2, cinv16, bf, ph1, wr2p, bl2p)
    return hg[:, :DOUT], lsm[:, :DOUT]


# speedup vs baseline: 11.0073x; 1.1659x over previous
import functools

import jax
import jax.numpy as jnp
from jax import lax
from jax.experimental import pallas as pl
from jax.experimental.pallas import tpu as pltpu
from jax.experimental.pallas import tpu_sc as plsc

N = 10000
E = 320000
DIN = 128
DH = 128
DOUT = 2
G = 64

NC = 2
NS = 16
NW = NC * NS
EPW = E // NW
RPT = N // NS
DC = 144
DQ = 16
KC1 = 50
KCQ = 125

BN = 400
NB = N // BN


def _make_seg_sum(D, kc):
    nch = EPW // kc
    mesh = plsc.VectorSubcoreMesh(core_axis_name="core", subcore_axis_name="subcore")

    def body(vals_hbm, src_hbm, dst_hbm, zeros_hbm, out_hbm,
             src_v, dst_v, rows0, rows1, acc_sh, sem0, sem1):
        c = lax.axis_index("core")
        s = lax.axis_index("subcore")
        wid = c * NS + s
        pltpu.sync_copy(src_hbm.at[wid], src_v)
        pltpu.sync_copy(dst_hbm.at[wid], dst_v)
        pltpu.sync_copy(zeros_hbm, acc_sh.at[pl.ds(s * RPT, RPT)])
        plsc.subcore_barrier()

        pltpu.async_copy(vals_hbm.at[src_v.at[0]], rows0, sem0)

        @pl.loop(0, nch, step=2)
        def _(j):
            pltpu.async_copy(vals_hbm.at[src_v.at[j + 1]], rows1, sem1)
            pltpu.make_async_copy(vals_hbm.at[pl.ds(0, kc)], rows0, sem0).wait()
            pltpu.sync_copy(rows0, acc_sh.at[dst_v.at[j]], add=True)

            @pl.when(j + 2 < nch)
            def _():
                pltpu.async_copy(vals_hbm.at[src_v.at[j + 2]], rows0, sem0)

            pltpu.make_async_copy(vals_hbm.at[pl.ds(0, kc)], rows1, sem1).wait()
            pltpu.sync_copy(rows1, acc_sh.at[dst_v.at[j + 1]], add=True)

        plsc.subcore_barrier()
        pltpu.sync_copy(acc_sh.at[pl.ds(s * RPT, RPT)],
                        out_hbm.at[c, pl.ds(s * RPT, RPT)])

    return pl.kernel(
        body,
        out_type=jax.ShapeDtypeStruct((NC, N, D), jnp.float32),
        mesh=mesh,
        compiler_params=pltpu.CompilerParams(use_tc_tiling_on_sc=False),
        scratch_types=[
            pltpu.VMEM((nch, kc), jnp.int32),
            pltpu.VMEM((nch, kc), jnp.int32),
            pltpu.VMEM((kc, D), jnp.float32),
            pltpu.VMEM((kc, D), jnp.float32),
            pltpu.VMEM_SHARED((N, D), jnp.float32),
            pltpu.SemaphoreType.DMA,
            pltpu.SemaphoreType.DMA,
        ],
    )


_make_seg_sum = functools.cache(_make_seg_sum)


def _onehot(bf_ref):
    gi = lax.broadcasted_iota(jnp.int32, (G, BN), 0).astype(jnp.float32)
    b = jnp.broadcast_to(bf_ref[0], (G, BN))
    return jnp.where(gi == b, 1.0, 0.0).astype(jnp.float32)


def _tc1_body(agg_ref, x_ref, bf_ref, wl1_ref, bl1_ref, wr1_ref,
              wl2p_ref, p_ref, cinv_ref, ph1_ref):
    i = pl.program_id(0)
    aa = agg_ref[0] + agg_ref[1]
    feats = aa[:, :DIN]
    cnt = aa[:, DIN:DIN + 1]
    cinv = 1.0 / jnp.maximum(cnt, 1.0)
    mean = feats * cinv
    h1 = jnp.dot(mean, wl1_ref[...], preferred_element_type=jnp.float32)
    h1 = h1 + bl1_ref[...]
    h1 = h1 + jnp.dot(x_ref[...], wr1_ref[...], preferred_element_type=jnp.float32)
    h1 = jnp.maximum(h1, 0.0)
    p_ref[...] = jnp.dot(h1, wl2p_ref[...], preferred_element_type=jnp.float32)
    cinv_ref[...] = jnp.broadcast_to(cinv, (BN, DQ))
    oh = _onehot(bf_ref)

    @pl.when(i == 0)
    def _():
        ph1_ref[...] = jnp.zeros_like(ph1_ref)

    ph1_ref[...] += jnp.dot(oh, h1, preferred_element_type=jnp.float32)


def _tc1(agg, x, bf, wl1, bl1, wr1, wl2p):
    return pl.pallas_call(
        _tc1_body,
        grid=(NB,),
        in_specs=[
            pl.BlockSpec((NC, BN, DC), lambda i: (0, i, 0)),
            pl.BlockSpec((BN, DIN), lambda i: (i, 0)),
            pl.BlockSpec((1, 1, BN), lambda i: (i, 0, 0)),
            pl.BlockSpec((DIN, DH), lambda i: (0, 0)),
            pl.BlockSpec((1, DH), lambda i: (0, 0)),
            pl.BlockSpec((DIN, DH), lambda i: (0, 0)),
            pl.BlockSpec((DH, DQ), lambda i: (0, 0)),
        ],
        out_specs=[
            pl.BlockSpec((BN, DQ), lambda i: (i, 0)),
            pl.BlockSpec((BN, DQ), lambda i: (i, 0)),
            pl.BlockSpec((G, DH), lambda i: (0, 0)),
        ],
        out_shape=[
            jax.ShapeDtypeStruct((N, DQ), jnp.float32),
            jax.ShapeDtypeStruct((N, DQ), jnp.float32),
            jax.ShapeDtypeStruct((G, DH), jnp.float32),
        ],
    )(agg, x, bf, wl1, bl1, wr1, wl2p)


def _tc2_body(q_ref, cinv_ref, bf_ref, ph1_ref, wr2p_ref, bl2p_ref,
              hg_ref, lsm_ref, pq_acc, gc_acc):
    i = pl.program_id(0)

    @pl.when(i == 0)
    def _():
        pq_acc[...] = jnp.zeros_like(pq_acc)
        gc_acc[...] = jnp.zeros_like(gc_acc)

    qq = (q_ref[0] + q_ref[1]) * cinv_ref[...]
    oh = _onehot(bf_ref)
    pq_acc[...] += jnp.dot(oh, qq, preferred_element_type=jnp.float32)
    gc_acc[...] += jnp.dot(oh, jnp.ones((BN, DQ), jnp.float32),
                           preferred_element_type=jnp.float32)

    @pl.when(i == NB - 1)
    def _():
        gc = gc_acc[:, 0:1]
        num = (pq_acc[...]
               + jnp.dot(ph1_ref[...], wr2p_ref[...],
                         preferred_element_type=jnp.float32)
               + gc * bl2p_ref[...])
        hg = num / jnp.maximum(gc, 1.0)
        lanes = lax.broadcasted_iota(jnp.int32, (G, DQ), 1)
        msk = lanes < DOUT
        mx = jnp.max(jnp.where(msk, hg, -1e30), axis=1, keepdims=True)
        sh = hg - mx
        e = jnp.where(msk, jnp.exp(sh), 0.0)
        se = jnp.sum(e, axis=1, keepdims=True)
        hg_ref[...] = hg
        lsm_ref[...] = sh - jnp.log(se)


def _tc2(q, cinv, bf, ph1, wr2p, bl2p):
    return pl.pallas_call(
        _tc2_body,
        grid=(NB,),
        in_specs=[
            pl.BlockSpec((NC, BN, DQ), lambda i: (0, i, 0)),
            pl.BlockSpec((BN, DQ), lambda i: (i, 0)),
            pl.BlockSpec((1, 1, BN), lambda i: (i, 0, 0)),
            pl.BlockSpec((G, DH), lambda i: (0, 0)),
            pl.BlockSpec((DH, DQ), lambda i: (0, 0)),
            pl.BlockSpec((1, DQ), lambda i: (0, 0)),
        ],
        out_specs=[
            pl.BlockSpec((G, DQ), lambda i: (0, 0)),
            pl.BlockSpec((G, DQ), lambda i: (0, 0)),
        ],
        out_shape=[
            jax.ShapeDtypeStruct((G, DQ), jnp.float32),
            jax.ShapeDtypeStruct((G, DQ), jnp.float32),
        ],
        scratch_shapes=[
            pltpu.VMEM((G, DQ), jnp.float32),
            pltpu.VMEM((G, DQ), jnp.float32),
        ],
    )(q, cinv, bf, ph1, wr2p, bl2p)


def kernel(x, edge_index, batch, W_l1, b_l1, W_r1, W_l2, b_l2, W_r2):
    f32 = jnp.float32
    xc = jnp.concatenate(
        [x, jnp.ones((N, 1), f32), jnp.zeros((N, DC - DIN - 1), f32)], axis=1)
    src1 = edge_index[0].reshape(NW, EPW // KC1, KC1)
    dst1 = edge_index[1].reshape(NW, EPW // KC1, KC1)

    part1 = _make_seg_sum(DC, KC1)(xc, src1, dst1, jnp.zeros((RPT, DC), f32))

    bf = batch.astype(f32).reshape(NB, 1, BN)
    wl2p = jnp.pad(W_l2, ((0, 0), (0, DQ - DOUT)))
    p, cinv16, ph1 = _tc1(part1, x, bf, W_l1, b_l1.reshape(1, DH),
                          W_r1, wl2p)

    srcq = edge_index[0].reshape(NW, EPW // KCQ, KCQ)
    dstq = edge_index[1].reshape(NW, EPW // KCQ, KCQ)
    part2 = _make_seg_sum(DQ, KCQ)(p, srcq, dstq, jnp.zeros((RPT, DQ), f32))

    wr2p = jnp.pad(W_r2, ((0, 0), (0, DQ - DOUT)))
    bl2p = jnp.pad(b_l2, (0, DQ - DOUT)).reshape(1, DQ)
    hg, lsm = _tc2(part2, cinv16, bf, ph1, wr2p, bl2p)
    return hg[:, :DOUT], lsm[:, :DOUT]


# speedup vs baseline: 12.0055x; 1.0907x over previous
import functools

import jax
import jax.numpy as jnp
from jax import lax
from jax.experimental import pallas as pl
from jax.experimental.pallas import tpu as pltpu
from jax.experimental.pallas import tpu_sc as plsc

N = 10000
E = 320000
DIN = 128
DH = 128
DOUT = 2
G = 64

NC = 2
NS = 16
NW = NC * NS
EPW = E // NW
RPT = N // NS
DCNT = 16
DQ = 16
KC1 = 50
KCQ = 125

BN = 400
NB = N // BN


def _make_seg_sum(D, kc):
    nch = EPW // kc
    mesh = plsc.VectorSubcoreMesh(core_axis_name="core", subcore_axis_name="subcore")

    def body(vals_hbm, src_hbm, dst_hbm, zeros_hbm, out_hbm,
             src_v, dst_v, rows0, rows1, acc_sh, sem0, sem1):
        c = lax.axis_index("core")
        s = lax.axis_index("subcore")
        wid = c * NS + s
        pltpu.sync_copy(src_hbm.at[wid], src_v)
        pltpu.sync_copy(dst_hbm.at[wid], dst_v)
        pltpu.sync_copy(zeros_hbm, acc_sh.at[pl.ds(s * RPT, RPT)])
        plsc.subcore_barrier()

        pltpu.async_copy(vals_hbm.at[src_v.at[0]], rows0, sem0)

        @pl.loop(0, nch, step=2)
        def _(j):
            pltpu.async_copy(vals_hbm.at[src_v.at[j + 1]], rows1, sem1)
            pltpu.make_async_copy(vals_hbm.at[pl.ds(0, kc)], rows0, sem0).wait()
            pltpu.sync_copy(rows0, acc_sh.at[dst_v.at[j]], add=True)

            @pl.when(j + 2 < nch)
            def _():
                pltpu.async_copy(vals_hbm.at[src_v.at[j + 2]], rows0, sem0)

            pltpu.make_async_copy(vals_hbm.at[pl.ds(0, kc)], rows1, sem1).wait()
            pltpu.sync_copy(rows1, acc_sh.at[dst_v.at[j + 1]], add=True)

        plsc.subcore_barrier()
        pltpu.sync_copy(acc_sh.at[pl.ds(s * RPT, RPT)],
                        out_hbm.at[c, pl.ds(s * RPT, RPT)])

    return pl.kernel(
        body,
        out_type=jax.ShapeDtypeStruct((NC, N, D), jnp.float32),
        mesh=mesh,
        compiler_params=pltpu.CompilerParams(use_tc_tiling_on_sc=False),
        scratch_types=[
            pltpu.VMEM((nch, kc), jnp.int32),
            pltpu.VMEM((nch, kc), jnp.int32),
            pltpu.VMEM((kc, D), jnp.float32),
            pltpu.VMEM((kc, D), jnp.float32),
            pltpu.VMEM_SHARED((N, D), jnp.float32),
            pltpu.SemaphoreType.DMA,
            pltpu.SemaphoreType.DMA,
        ],
    )


_make_seg_sum = functools.cache(_make_seg_sum)


def _make_seg_sum1(kc):
    nch = EPW // kc
    mesh = plsc.VectorSubcoreMesh(core_axis_name="core", subcore_axis_name="subcore")

    def body(vals_hbm, src_hbm, dst_hbm, zf_hbm, zc_hbm, ones_hbm,
             out_hbm, outc_hbm,
             src_v, dst_v, rows0, rows1, ones_v, acc_sh, cnt_sh, sem0, sem1):
        c = lax.axis_index("core")
        s = lax.axis_index("subcore")
        wid = c * NS + s
        pltpu.sync_copy(src_hbm.at[wid], src_v)
        pltpu.sync_copy(dst_hbm.at[wid], dst_v)
        pltpu.sync_copy(ones_hbm, ones_v)
        pltpu.sync_copy(zf_hbm, acc_sh.at[pl.ds(s * RPT, RPT)])
        pltpu.sync_copy(zc_hbm, cnt_sh.at[pl.ds(s * RPT, RPT)])
        plsc.subcore_barrier()

        pltpu.async_copy(vals_hbm.at[src_v.at[0]], rows0, sem0)

        @pl.loop(0, nch, step=2)
        def _(j):
            pltpu.async_copy(vals_hbm.at[src_v.at[j + 1]], rows1, sem1)
            pltpu.make_async_copy(vals_hbm.at[pl.ds(0, kc)], rows0, sem0).wait()
            pltpu.sync_copy(rows0, acc_sh.at[dst_v.at[j]], add=True)
            pltpu.sync_copy(ones_v, cnt_sh.at[dst_v.at[j]], add=True)

            @pl.when(j + 2 < nch)
            def _():
                pltpu.async_copy(vals_hbm.at[src_v.at[j + 2]], rows0, sem0)

            pltpu.make_async_copy(vals_hbm.at[pl.ds(0, kc)], rows1, sem1).wait()
            pltpu.sync_copy(rows1, acc_sh.at[dst_v.at[j + 1]], add=True)
            pltpu.sync_copy(ones_v, cnt_sh.at[dst_v.at[j + 1]], add=True)

        plsc.subcore_barrier()
        pltpu.sync_copy(acc_sh.at[pl.ds(s * RPT, RPT)],
                        out_hbm.at[c, pl.ds(s * RPT, RPT)])
        pltpu.sync_copy(cnt_sh.at[pl.ds(s * RPT, RPT)],
                        outc_hbm.at[c, pl.ds(s * RPT, RPT)])

    return pl.kernel(
        body,
        out_type=(jax.ShapeDtypeStruct((NC, N, DIN), jnp.float32),
                  jax.ShapeDtypeStruct((NC, N, DCNT), jnp.float32)),
        mesh=mesh,
        compiler_params=pltpu.CompilerParams(use_tc_tiling_on_sc=False),
        scratch_types=[
            pltpu.VMEM((nch, kc), jnp.int32),
            pltpu.VMEM((nch, kc), jnp.int32),
            pltpu.VMEM((kc, DIN), jnp.float32),
            pltpu.VMEM((kc, DIN), jnp.float32),
            pltpu.VMEM((kc, DCNT), jnp.float32),
            pltpu.VMEM_SHARED((N, DIN), jnp.float32),
            pltpu.VMEM_SHARED((N, DCNT), jnp.float32),
            pltpu.SemaphoreType.DMA,
            pltpu.SemaphoreType.DMA,
        ],
    )


_make_seg_sum1 = functools.cache(_make_seg_sum1)


def _onehot(bf_ref):
    gi = lax.broadcasted_iota(jnp.int32, (G, BN), 0).astype(jnp.float32)
    b = jnp.broadcast_to(bf_ref[0], (G, BN))
    return jnp.where(gi == b, 1.0, 0.0).astype(jnp.float32)


def _tc1_body(agg_ref, cnt_ref, x_ref, bf_ref, wl1_ref, bl1_ref, wr1_ref,
              wl2p_ref, p_ref, cinv_ref, ph1_ref):
    i = pl.program_id(0)
    feats = agg_ref[0] + agg_ref[1]
    cnt = (cnt_ref[0] + cnt_ref[1])[:, 0:1]
    cinv = 1.0 / jnp.maximum(cnt, 1.0)
    mean = feats * cinv
    h1 = jnp.dot(mean, wl1_ref[...], preferred_element_type=jnp.float32)
    h1 = h1 + bl1_ref[...]
    h1 = h1 + jnp.dot(x_ref[...], wr1_ref[...], preferred_element_type=jnp.float32)
    h1 = jnp.maximum(h1, 0.0)
    p_ref[...] = jnp.dot(h1, wl2p_ref[...], preferred_element_type=jnp.float32)
    cinv_ref[...] = jnp.broadcast_to(cinv, (BN, DQ))
    oh = _onehot(bf_ref)

    @pl.when(i == 0)
    def _():
        ph1_ref[...] = jnp.zeros_like(ph1_ref)

    ph1_ref[...] += jnp.dot(oh, h1, preferred_element_type=jnp.float32)


def _tc1(agg, cnt, x, bf, wl1, bl1, wr1, wl2p):
    return pl.pallas_call(
        _tc1_body,
        grid=(NB,),
        in_specs=[
            pl.BlockSpec((NC, BN, DIN), lambda i: (0, i, 0)),
            pl.BlockSpec((NC, BN, DCNT), lambda i: (0, i, 0)),
            pl.BlockSpec((BN, DIN), lambda i: (i, 0)),
            pl.BlockSpec((1, 1, BN), lambda i: (i, 0, 0)),
            pl.BlockSpec((DIN, DH), lambda i: (0, 0)),
            pl.BlockSpec((1, DH), lambda i: (0, 0)),
            pl.BlockSpec((DIN, DH), lambda i: (0, 0)),
            pl.BlockSpec((DH, DQ), lambda i: (0, 0)),
        ],
        out_specs=[
            pl.BlockSpec((BN, DQ), lambda i: (i, 0)),
            pl.BlockSpec((BN, DQ), lambda i: (i, 0)),
            pl.BlockSpec((G, DH), lambda i: (0, 0)),
        ],
        out_shape=[
            jax.ShapeDtypeStruct((N, DQ), jnp.float32),
            jax.ShapeDtypeStruct((N, DQ), jnp.float32),
            jax.ShapeDtypeStruct((G, DH), jnp.float32),
        ],
    )(agg, cnt, x, bf, wl1, bl1, wr1, wl2p)


def _tc2_body(q_ref, cinv_ref, bf_ref, ph1_ref, wr2p_ref, bl2p_ref,
              hg_ref, lsm_ref, pq_acc, gc_acc):
    i = pl.program_id(0)

    @pl.when(i == 0)
    def _():
        pq_acc[...] = jnp.zeros_like(pq_acc)
        gc_acc[...] = jnp.zeros_like(gc_acc)

    qq = (q_ref[0] + q_ref[1]) * cinv_ref[...]
    oh = _onehot(bf_ref)
    pq_acc[...] += jnp.dot(oh, qq, preferred_element_type=jnp.float32)
    gc_acc[...] += jnp.dot(oh, jnp.ones((BN, DQ), jnp.float32),
                           preferred_element_type=jnp.float32)

    @pl.when(i == NB - 1)
    def _():
        gc = gc_acc[:, 0:1]
        num = (pq_acc[...]
               + jnp.dot(ph1_ref[...], wr2p_ref[...],
                         preferred_element_type=jnp.float32)
               + gc * bl2p_ref[...])
        hg = num / jnp.maximum(gc, 1.0)
        lanes = lax.broadcasted_iota(jnp.int32, (G, DQ), 1)
        msk = lanes < DOUT
        mx = jnp.max(jnp.where(msk, hg, -1e30), axis=1, keepdims=True)
        sh = hg - mx
        e = jnp.where(msk, jnp.exp(sh), 0.0)
        se = jnp.sum(e, axis=1, keepdims=True)
        hg_ref[...] = hg
        lsm_ref[...] = sh - jnp.log(se)


def _tc2(q, cinv, bf, ph1, wr2p, bl2p):
    return pl.pallas_call(
        _tc2_body,
        grid=(NB,),
        in_specs=[
            pl.BlockSpec((NC, BN, DQ), lambda i: (0, i, 0)),
            pl.BlockSpec((BN, DQ), lambda i: (i, 0)),
            pl.BlockSpec((1, 1, BN), lambda i: (i, 0, 0)),
            pl.BlockSpec((G, DH), lambda i: (0, 0)),
            pl.BlockSpec((DH, DQ), lambda i: (0, 0)),
            pl.BlockSpec((1, DQ), lambda i: (0, 0)),
        ],
        out_specs=[
            pl.BlockSpec((G, DQ), lambda i: (0, 0)),
            pl.BlockSpec((G, DQ), lambda i: (0, 0)),
        ],
        out_shape=[
            jax.ShapeDtypeStruct((G, DQ), jnp.float32),
            jax.ShapeDtypeStruct((G, DQ), jnp.float32),
        ],
        scratch_shapes=[
            pltpu.VMEM((G, DQ), jnp.float32),
            pltpu.VMEM((G, DQ), jnp.float32),
        ],
    )(q, cinv, bf, ph1, wr2p, bl2p)


def kernel(x, edge_index, batch, W_l1, b_l1, W_r1, W_l2, b_l2, W_r2):
    f32 = jnp.float32
    src1 = edge_index[0].reshape(NW, EPW // KC1, KC1)
    dst1 = edge_index[1].reshape(NW, EPW // KC1, KC1)

    part1, cnt1 = _make_seg_sum1(KC1)(
        x, src1, dst1, jnp.zeros((RPT, DIN), f32), jnp.zeros((RPT, DCNT), f32),
        jnp.ones((KC1, DCNT), f32))

    bf = batch.astype(f32).reshape(NB, 1, BN)
    wl2p = jnp.pad(W_l2, ((0, 0), (0, DQ - DOUT)))
    p, cinv16, ph1 = _tc1(part1, cnt1, x, bf, W_l1, b_l1.reshape(1, DH),
                          W_r1, wl2p)

    srcq = edge_index[0].reshape(NW, EPW // KCQ, KCQ)
    dstq = edge_index[1].reshape(NW, EPW // KCQ, KCQ)
    part2 = _make_seg_sum(DQ, KCQ)(p, srcq, dstq, jnp.zeros((RPT, DQ), f32))

    wr2p = jnp.pad(W_r2, ((0, 0), (0, DQ - DOUT)))
    bl2p = jnp.pad(b_l2, (0, DQ - DOUT)).reshape(1, DQ)
    hg, lsm = _tc2(part2, cinv16, bf, ph1, wr2p, bl2p)
    return hg[:, :DOUT], lsm[:, :DOUT]


# speedup vs baseline: 12.9710x; 1.0804x over previous
import functools

import jax
import jax.numpy as jnp
from jax import lax
from jax.experimental import pallas as pl
from jax.experimental.pallas import tpu as pltpu
from jax.experimental.pallas import tpu_sc as plsc

N = 10000
E = 320000
DIN = 128
DH = 128
DOUT = 2
G = 64

NC = 2
NS = 16
NW = NC * NS
EPW = E // NW
RPT = N // NS
DCNT = 16
DQ = 16
KC1 = 50
KCQ = 125

BN = 2000
NB = N // BN


def _make_seg_sum(D, kc):
    nch = EPW // kc
    mesh = plsc.VectorSubcoreMesh(core_axis_name="core", subcore_axis_name="subcore")

    def body(vals_hbm, src_hbm, dst_hbm, zeros_hbm, out_hbm,
             src_v, dst_v, rows0, rows1, acc_sh, sem0, sem1):
        c = lax.axis_index("core")
        s = lax.axis_index("subcore")
        wid = c * NS + s
        pltpu.sync_copy(src_hbm.at[wid], src_v)
        pltpu.sync_copy(dst_hbm.at[wid], dst_v)
        pltpu.sync_copy(zeros_hbm, acc_sh.at[pl.ds(s * RPT, RPT)])
        plsc.subcore_barrier()

        pltpu.async_copy(vals_hbm.at[src_v.at[0]], rows0, sem0)

        @pl.loop(0, nch, step=2)
        def _(j):
            pltpu.async_copy(vals_hbm.at[src_v.at[j + 1]], rows1, sem1)
            pltpu.make_async_copy(vals_hbm.at[pl.ds(0, kc)], rows0, sem0).wait()
            pltpu.sync_copy(rows0, acc_sh.at[dst_v.at[j]], add=True)

            @pl.when(j + 2 < nch)
            def _():
                pltpu.async_copy(vals_hbm.at[src_v.at[j + 2]], rows0, sem0)

            pltpu.make_async_copy(vals_hbm.at[pl.ds(0, kc)], rows1, sem1).wait()
            pltpu.sync_copy(rows1, acc_sh.at[dst_v.at[j + 1]], add=True)

        plsc.subcore_barrier()
        pltpu.sync_copy(acc_sh.at[pl.ds(s * RPT, RPT)],
                        out_hbm.at[c, pl.ds(s * RPT, RPT)])

    return pl.kernel(
        body,
        out_type=jax.ShapeDtypeStruct((NC, N, D), jnp.float32),
        mesh=mesh,
        compiler_params=pltpu.CompilerParams(use_tc_tiling_on_sc=False),
        scratch_types=[
            pltpu.VMEM((nch, kc), jnp.int32),
            pltpu.VMEM((nch, kc), jnp.int32),
            pltpu.VMEM((kc, D), jnp.float32),
            pltpu.VMEM((kc, D), jnp.float32),
            pltpu.VMEM_SHARED((N, D), jnp.float32),
            pltpu.SemaphoreType.DMA,
            pltpu.SemaphoreType.DMA,
        ],
    )


_make_seg_sum = functools.cache(_make_seg_sum)


def _make_seg_sum1(kc):
    nch = EPW // kc
    mesh = plsc.VectorSubcoreMesh(core_axis_name="core", subcore_axis_name="subcore")

    def body(vals_hbm, src_hbm, dst_hbm, zf_hbm, zc_hbm, ones_hbm,
             out_hbm, outc_hbm,
             src_v, dst_v, rows0, rows1, ones_v, acc_sh, cnt_sh, sem0, sem1):
        c = lax.axis_index("core")
        s = lax.axis_index("subcore")
        wid = c * NS + s
        pltpu.sync_copy(src_hbm.at[wid], src_v)
        pltpu.sync_copy(dst_hbm.at[wid], dst_v)
        pltpu.sync_copy(ones_hbm, ones_v)
        pltpu.sync_copy(zf_hbm, acc_sh.at[pl.ds(s * RPT, RPT)])
        pltpu.sync_copy(zc_hbm, cnt_sh.at[pl.ds(s * RPT, RPT)])
        plsc.subcore_barrier()

        pltpu.async_copy(vals_hbm.at[src_v.at[0]], rows0, sem0)

        @pl.loop(0, nch, step=2)
        def _(j):
            pltpu.async_copy(vals_hbm.at[src_v.at[j + 1]], rows1, sem1)
            pltpu.make_async_copy(vals_hbm.at[pl.ds(0, kc)], rows0, sem0).wait()
            pltpu.sync_copy(rows0, acc_sh.at[dst_v.at[j]], add=True)
            pltpu.sync_copy(ones_v, cnt_sh.at[dst_v.at[j]], add=True)

            @pl.when(j + 2 < nch)
            def _():
                pltpu.async_copy(vals_hbm.at[src_v.at[j + 2]], rows0, sem0)

            pltpu.make_async_copy(vals_hbm.at[pl.ds(0, kc)], rows1, sem1).wait()
            pltpu.sync_copy(rows1, acc_sh.at[dst_v.at[j + 1]], add=True)
            pltpu.sync_copy(ones_v, cnt_sh.at[dst_v.at[j + 1]], add=True)

        plsc.subcore_barrier()
        pltpu.sync_copy(acc_sh.at[pl.ds(s * RPT, RPT)],
                        out_hbm.at[c, pl.ds(s * RPT, RPT)])
        pltpu.sync_copy(cnt_sh.at[pl.ds(s * RPT, RPT)],
                        outc_hbm.at[c, pl.ds(s * RPT, RPT)])

    return pl.kernel(
        body,
        out_type=(jax.ShapeDtypeStruct((NC, N, DIN), jnp.float32),
                  jax.ShapeDtypeStruct((NC, N, DCNT), jnp.float32)),
        mesh=mesh,
        compiler_params=pltpu.CompilerParams(use_tc_tiling_on_sc=False),
        scratch_types=[
            pltpu.VMEM((nch, kc), jnp.int32),
            pltpu.VMEM((nch, kc), jnp.int32),
            pltpu.VMEM((kc, DIN), jnp.float32),
            pltpu.VMEM((kc, DIN), jnp.float32),
            pltpu.VMEM((kc, DCNT), jnp.float32),
            pltpu.VMEM_SHARED((N, DIN), jnp.float32),
            pltpu.VMEM_SHARED((N, DCNT), jnp.float32),
            pltpu.SemaphoreType.DMA,
            pltpu.SemaphoreType.DMA,
        ],
    )


_make_seg_sum1 = functools.cache(_make_seg_sum1)


def _onehot(bf_ref):
    gi = lax.broadcasted_iota(jnp.int32, (G, BN), 0).astype(jnp.float32)
    b = jnp.broadcast_to(bf_ref[0], (G, BN))
    return jnp.where(gi == b, 1.0, 0.0).astype(jnp.float32)


def _tc1_body(agg_ref, cnt_ref, x_ref, bf_ref, wl1_ref, bl1_ref, wr1_ref,
              wl2p_ref, p_ref, cinv_ref, ph1_ref):
    i = pl.program_id(0)
    feats = agg_ref[0] + agg_ref[1]
    cnt = (cnt_ref[0] + cnt_ref[1])[:, 0:1]
    cinv = 1.0 / jnp.maximum(cnt, 1.0)
    mean = feats * cinv
    h1 = jnp.dot(mean, wl1_ref[...], preferred_element_type=jnp.float32)
    h1 = h1 + bl1_ref[...]
    h1 = h1 + jnp.dot(x_ref[...], wr1_ref[...], preferred_element_type=jnp.float32)
    h1 = jnp.maximum(h1, 0.0)
    p_ref[...] = jnp.dot(h1, wl2p_ref[...], preferred_element_type=jnp.float32)
    cinv_ref[...] = jnp.broadcast_to(cinv, (BN, DQ))
    oh = _onehot(bf_ref)

    @pl.when(i == 0)
    def _():
        ph1_ref[...] = jnp.zeros_like(ph1_ref)

    ph1_ref[...] += jnp.dot(oh, h1, preferred_element_type=jnp.float32)


def _tc1(agg, cnt, x, bf, wl1, bl1, wr1, wl2p):
    return pl.pallas_call(
        _tc1_body,
        grid=(NB,),
        in_specs=[
            pl.BlockSpec((NC, BN, DIN), lambda i: (0, i, 0)),
            pl.BlockSpec((NC, BN, DCNT), lambda i: (0, i, 0)),
            pl.BlockSpec((BN, DIN), lambda i: (i, 0)),
            pl.BlockSpec((1, 1, BN), lambda i: (i, 0, 0)),
            pl.BlockSpec((DIN, DH), lambda i: (0, 0)),
            pl.BlockSpec((1, DH), lambda i: (0, 0)),
            pl.BlockSpec((DIN, DH), lambda i: (0, 0)),
            pl.BlockSpec((DH, DQ), lambda i: (0, 0)),
        ],
        out_specs=[
            pl.BlockSpec((BN, DQ), lambda i: (i, 0)),
            pl.BlockSpec((BN, DQ), lambda i: (i, 0)),
            pl.BlockSpec((G, DH), lambda i: (0, 0)),
        ],
        out_shape=[
            jax.ShapeDtypeStruct((N, DQ), jnp.float32),
            jax.ShapeDtypeStruct((N, DQ), jnp.float32),
            jax.ShapeDtypeStruct((G, DH), jnp.float32),
        ],
    )(agg, cnt, x, bf, wl1, bl1, wr1, wl2p)


def _tc2_body(q_ref, cinv_ref, bf_ref, ph1_ref, wr2p_ref, bl2p_ref,
              hg_ref, lsm_ref, pq_acc, gc_acc):
    i = pl.program_id(0)

    @pl.when(i == 0)
    def _():
        pq_acc[...] = jnp.zeros_like(pq_acc)
        gc_acc[...] = jnp.zeros_like(gc_acc)

    qq = (q_ref[0] + q_ref[1]) * cinv_ref[...]
    oh = _onehot(bf_ref)
    pq_acc[...] += jnp.dot(oh, qq, preferred_element_type=jnp.float32)
    gc_acc[...] += jnp.dot(oh, jnp.ones((BN, DQ), jnp.float32),
                           preferred_element_type=jnp.float32)

    @pl.when(i == NB - 1)
    def _():
        gc = gc_acc[:, 0:1]
        num = (pq_acc[...]
               + jnp.dot(ph1_ref[...], wr2p_ref[...],
                         preferred_element_type=jnp.float32)
               + gc * bl2p_ref[...])
        hg = num / jnp.maximum(gc, 1.0)
        lanes = lax.broadcasted_iota(jnp.int32, (G, DQ), 1)
        msk = lanes < DOUT
        mx = jnp.max(jnp.where(msk, hg, -1e30), axis=1, keepdims=True)
        sh = hg - mx
        e = jnp.where(msk, jnp.exp(sh), 0.0)
        se = jnp.sum(e, axis=1, keepdims=True)
        hg_ref[...] = hg
        lsm_ref[...] = sh - jnp.log(se)


def _tc2(q, cinv, bf, ph1, wr2p, bl2p):
    return pl.pallas_call(
        _tc2_body,
        grid=(NB,),
        in_specs=[
            pl.BlockSpec((NC, BN, DQ), lambda i: (0, i, 0)),
            pl.BlockSpec((BN, DQ), lambda i: (i, 0)),
            pl.BlockSpec((1, 1, BN), lambda i: (i, 0, 0)),
            pl.BlockSpec((G, DH), lambda i: (0, 0)),
            pl.BlockSpec((DH, DQ), lambda i: (0, 0)),
            pl.BlockSpec((1, DQ), lambda i: (0, 0)),
        ],
        out_specs=[
            pl.BlockSpec((G, DQ), lambda i: (0, 0)),
            pl.BlockSpec((G, DQ), lambda i: (0, 0)),
        ],
        out_shape=[
            jax.ShapeDtypeStruct((G, DQ), jnp.float32),
            jax.ShapeDtypeStruct((G, DQ), jnp.float32),
        ],
        scratch_shapes=[
            pltpu.VMEM((G, DQ), jnp.float32),
            pltpu.VMEM((G, DQ), jnp.float32),
        ],
    )(q, cinv, bf, ph1, wr2p, bl2p)


def kernel(x, edge_index, batch, W_l1, b_l1, W_r1, W_l2, b_l2, W_r2):
    f32 = jnp.float32
    src1 = edge_index[0].reshape(NW, EPW // KC1, KC1)
    dst1 = edge_index[1].reshape(NW, EPW // KC1, KC1)

    part1, cnt1 = _make_seg_sum1(KC1)(
        x, src1, dst1, jnp.zeros((RPT, DIN), f32), jnp.zeros((RPT, DCNT), f32),
        jnp.ones((KC1, DCNT), f32))

    bf = batch.astype(f32).reshape(NB, 1, BN)
    wl2p = jnp.pad(W_l2, ((0, 0), (0, DQ - DOUT)))
    p, cinv16, ph1 = _tc1(part1, cnt1, x, bf, W_l1, b_l1.reshape(1, DH),
                          W_r1, wl2p)

    srcq = edge_index[0].reshape(NW, EPW // KCQ, KCQ)
    dstq = edge_index[1].reshape(NW, EPW // KCQ, KCQ)
    part2 = _make_seg_sum(DQ, KCQ)(p, srcq, dstq, jnp.zeros((RPT, DQ), f32))

    wr2p = jnp.pad(W_r2, ((0, 0), (0, DQ - DOUT)))
    bl2p = jnp.pad(b_l2, (0, DQ - DOUT)).reshape(1, DQ)
    hg, lsm = _tc2(part2, cinv16, bf, ph1, wr2p, bl2p)
    return hg[:, :DOUT], lsm[:, :DOUT]


# speedup vs baseline: 12.9958x; 1.0019x over previous
import functools

import jax
import jax.numpy as jnp
from jax import lax
from jax.experimental import pallas as pl
from jax.experimental.pallas import tpu as pltpu
from jax.experimental.pallas import tpu_sc as plsc

N = 10000
E = 320000
DIN = 128
DH = 128
DOUT = 2
G = 64

NC = 2
NS = 16
NW = NC * NS
EPW = E // NW
RPT = N // NS
DCNT = 16
DQ = 16
KC1 = 50
KCQ = 125

BN = 10000
NB = N // BN


def _make_seg_sum(D, kc):
    nch = EPW // kc
    mesh = plsc.VectorSubcoreMesh(core_axis_name="core", subcore_axis_name="subcore")

    def body(vals_hbm, src_hbm, dst_hbm, zeros_hbm, out_hbm,
             src_v, dst_v, rows0, rows1, acc_sh, sem0, sem1):
        c = lax.axis_index("core")
        s = lax.axis_index("subcore")
        wid = c * NS + s
        pltpu.sync_copy(src_hbm.at[wid], src_v)
        pltpu.sync_copy(dst_hbm.at[wid], dst_v)
        pltpu.sync_copy(zeros_hbm, acc_sh.at[pl.ds(s * RPT, RPT)])
        plsc.subcore_barrier()

        pltpu.async_copy(vals_hbm.at[src_v.at[0]], rows0, sem0)

        @pl.loop(0, nch, step=2)
        def _(j):
            pltpu.async_copy(vals_hbm.at[src_v.at[j + 1]], rows1, sem1)
            pltpu.make_async_copy(vals_hbm.at[pl.ds(0, kc)], rows0, sem0).wait()
            pltpu.sync_copy(rows0, acc_sh.at[dst_v.at[j]], add=True)

            @pl.when(j + 2 < nch)
            def _():
                pltpu.async_copy(vals_hbm.at[src_v.at[j + 2]], rows0, sem0)

            pltpu.make_async_copy(vals_hbm.at[pl.ds(0, kc)], rows1, sem1).wait()
            pltpu.sync_copy(rows1, acc_sh.at[dst_v.at[j + 1]], add=True)

        plsc.subcore_barrier()
        pltpu.sync_copy(acc_sh.at[pl.ds(s * RPT, RPT)],
                        out_hbm.at[c, pl.ds(s * RPT, RPT)])

    return pl.kernel(
        body,
        out_type=jax.ShapeDtypeStruct((NC, N, D), jnp.float32),
        mesh=mesh,
        compiler_params=pltpu.CompilerParams(use_tc_tiling_on_sc=False),
        scratch_types=[
            pltpu.VMEM((nch, kc), jnp.int32),
            pltpu.VMEM((nch, kc), jnp.int32),
            pltpu.VMEM((kc, D), jnp.float32),
            pltpu.VMEM((kc, D), jnp.float32),
            pltpu.VMEM_SHARED((N, D), jnp.float32),
            pltpu.SemaphoreType.DMA,
            pltpu.SemaphoreType.DMA,
        ],
    )


_make_seg_sum = functools.cache(_make_seg_sum)


def _make_seg_sum1(kc):
    nch = EPW // kc
    mesh = plsc.VectorSubcoreMesh(core_axis_name="core", subcore_axis_name="subcore")

    def body(vals_hbm, src_hbm, dst_hbm, zf_hbm, zc_hbm, ones_hbm,
             out_hbm, outc_hbm,
             src_v, dst_v, rows0, rows1, ones_v, acc_sh, cnt_sh, sem0, sem1):
        c = lax.axis_index("core")
        s = lax.axis_index("subcore")
        wid = c * NS + s
        pltpu.sync_copy(src_hbm.at[wid], src_v)
        pltpu.sync_copy(dst_hbm.at[wid], dst_v)
        pltpu.sync_copy(ones_hbm, ones_v)
        pltpu.sync_copy(zf_hbm, acc_sh.at[pl.ds(s * RPT, RPT)])
        pltpu.sync_copy(zc_hbm, cnt_sh.at[pl.ds(s * RPT, RPT)])
        plsc.subcore_barrier()

        pltpu.async_copy(vals_hbm.at[src_v.at[0]], rows0, sem0)

        @pl.loop(0, nch, step=2)
        def _(j):
            pltpu.async_copy(vals_hbm.at[src_v.at[j + 1]], rows1, sem1)
            pltpu.make_async_copy(vals_hbm.at[pl.ds(0, kc)], rows0, sem0).wait()
            pltpu.sync_copy(rows0, acc_sh.at[dst_v.at[j]], add=True)
            pltpu.sync_copy(ones_v, cnt_sh.at[dst_v.at[j]], add=True)

            @pl.when(j + 2 < nch)
            def _():
                pltpu.async_copy(vals_hbm.at[src_v.at[j + 2]], rows0, sem0)

            pltpu.make_async_copy(vals_hbm.at[pl.ds(0, kc)], rows1, sem1).wait()
            pltpu.sync_copy(rows1, acc_sh.at[dst_v.at[j + 1]], add=True)
            pltpu.sync_copy(ones_v, cnt_sh.at[dst_v.at[j + 1]], add=True)

        plsc.subcore_barrier()
        pltpu.sync_copy(acc_sh.at[pl.ds(s * RPT, RPT)],
                        out_hbm.at[c, pl.ds(s * RPT, RPT)])
        pltpu.sync_copy(cnt_sh.at[pl.ds(s * RPT, RPT)],
                        outc_hbm.at[c, pl.ds(s * RPT, RPT)])

    return pl.kernel(
        body,
        out_type=(jax.ShapeDtypeStruct((NC, N, DIN), jnp.float32),
                  jax.ShapeDtypeStruct((NC, N, DCNT), jnp.float32)),
        mesh=mesh,
        compiler_params=pltpu.CompilerParams(use_tc_tiling_on_sc=False),
        scratch_types=[
            pltpu.VMEM((nch, kc), jnp.int32),
            pltpu.VMEM((nch, kc), jnp.int32),
            pltpu.VMEM((kc, DIN), jnp.float32),
            pltpu.VMEM((kc, DIN), jnp.float32),
            pltpu.VMEM((kc, DCNT), jnp.float32),
            pltpu.VMEM_SHARED((N, DIN), jnp.float32),
            pltpu.VMEM_SHARED((N, DCNT), jnp.float32),
            pltpu.SemaphoreType.DMA,
            pltpu.SemaphoreType.DMA,
        ],
    )


_make_seg_sum1 = functools.cache(_make_seg_sum1)


def _onehot(bf_ref):
    gi = lax.broadcasted_iota(jnp.int32, (G, BN), 0).astype(jnp.float32)
    b = jnp.broadcast_to(bf_ref[0], (G, BN))
    return jnp.where(gi == b, 1.0, 0.0).astype(jnp.float32)


def _tc1_body(agg_ref, cnt_ref, x_ref, bf_ref, wl1_ref, bl1_ref, wr1_ref,
              wl2p_ref, p_ref, cinv_ref, ph1_ref):
    i = pl.program_id(0)
    feats = agg_ref[0] + agg_ref[1]
    cnt = (cnt_ref[0] + cnt_ref[1])[:, 0:1]
    cinv = 1.0 / jnp.maximum(cnt, 1.0)
    mean = feats * cinv
    h1 = jnp.dot(mean, wl1_ref[...], preferred_element_type=jnp.float32)
    h1 = h1 + bl1_ref[...]
    h1 = h1 + jnp.dot(x_ref[...], wr1_ref[...], preferred_element_type=jnp.float32)
    h1 = jnp.maximum(h1, 0.0)
    p_ref[...] = jnp.dot(h1, wl2p_ref[...], preferred_element_type=jnp.float32)
    cinv_ref[...] = jnp.broadcast_to(cinv, (BN, DQ))
    oh = _onehot(bf_ref)

    @pl.when(i == 0)
    def _():
        ph1_ref[...] = jnp.zeros_like(ph1_ref)

    ph1_ref[...] += jnp.dot(oh, h1, preferred_element_type=jnp.float32)


def _tc1(agg, cnt, x, bf, wl1, bl1, wr1, wl2p):
    return pl.pallas_call(
        _tc1_body,
        grid=(NB,),
        in_specs=[
            pl.BlockSpec((NC, BN, DIN), lambda i: (0, i, 0)),
            pl.BlockSpec((NC, BN, DCNT), lambda i: (0, i, 0)),
            pl.BlockSpec((BN, DIN), lambda i: (i, 0)),
            pl.BlockSpec((1, 1, BN), lambda i: (i, 0, 0)),
            pl.BlockSpec((DIN, DH), lambda i: (0, 0)),
            pl.BlockSpec((1, DH), lambda i: (0, 0)),
            pl.BlockSpec((DIN, DH), lambda i: (0, 0)),
            pl.BlockSpec((DH, DQ), lambda i: (0, 0)),
        ],
        out_specs=[
            pl.BlockSpec((BN, DQ), lambda i: (i, 0)),
            pl.BlockSpec((BN, DQ), lambda i: (i, 0)),
            pl.BlockSpec((G, DH), lambda i: (0, 0)),
        ],
        out_shape=[
            jax.ShapeDtypeStruct((N, DQ), jnp.float32),
            jax.ShapeDtypeStruct((N, DQ), jnp.float32),
            jax.ShapeDtypeStruct((G, DH), jnp.float32),
        ],
    )(agg, cnt, x, bf, wl1, bl1, wr1, wl2p)


def _tc2_body(q_ref, cinv_ref, bf_ref, ph1_ref, wr2p_ref, bl2p_ref,
              hg_ref, lsm_ref, pq_acc, gc_acc):
    i = pl.program_id(0)

    @pl.when(i == 0)
    def _():
        pq_acc[...] = jnp.zeros_like(pq_acc)
        gc_acc[...] = jnp.zeros_like(gc_acc)

    qq = (q_ref[0] + q_ref[1]) * cinv_ref[...]
    oh = _onehot(bf_ref)
    pq_acc[...] += jnp.dot(oh, qq, preferred_element_type=jnp.float32)
    gc_acc[...] += jnp.dot(oh, jnp.ones((BN, DQ), jnp.float32),
                           preferred_element_type=jnp.float32)

    @pl.when(i == NB - 1)
    def _():
        gc = gc_acc[:, 0:1]
        num = (pq_acc[...]
               + jnp.dot(ph1_ref[...], wr2p_ref[...],
                         preferred_element_type=jnp.float32)
               + gc * bl2p_ref[...])
        hg = num / jnp.maximum(gc, 1.0)
        lanes = lax.broadcasted_iota(jnp.int32, (G, DQ), 1)
        msk = lanes < DOUT
        mx = jnp.max(jnp.where(msk, hg, -1e30), axis=1, keepdims=True)
        sh = hg - mx
        e = jnp.where(msk, jnp.exp(sh), 0.0)
        se = jnp.sum(e, axis=1, keepdims=True)
        hg_ref[...] = hg
        lsm_ref[...] = sh - jnp.log(se)


def _tc2(q, cinv, bf, ph1, wr2p, bl2p):
    return pl.pallas_call(
        _tc2_body,
        grid=(NB,),
        in_specs=[
            pl.BlockSpec((NC, BN, DQ), lambda i: (0, i, 0)),
            pl.BlockSpec((BN, DQ), lambda i: (i, 0)),
            pl.BlockSpec((1, 1, BN), lambda i: (i, 0, 0)),
            pl.BlockSpec((G, DH), lambda i: (0, 0)),
            pl.BlockSpec((DH, DQ), lambda i: (0, 0)),
            pl.BlockSpec((1, DQ), lambda i: (0, 0)),
        ],
        out_specs=[
            pl.BlockSpec((G, DQ), lambda i: (0, 0)),
            pl.BlockSpec((G, DQ), lambda i: (0, 0)),
        ],
        out_shape=[
            jax.ShapeDtypeStruct((G, DQ), jnp.float32),
            jax.ShapeDtypeStruct((G, DQ), jnp.float32),
        ],
        scratch_shapes=[
            pltpu.VMEM((G, DQ), jnp.float32),
            pltpu.VMEM((G, DQ), jnp.float32),
        ],
    )(q, cinv, bf, ph1, wr2p, bl2p)


def kernel(x, edge_index, batch, W_l1, b_l1, W_r1, W_l2, b_l2, W_r2):
    f32 = jnp.float32
    src1 = edge_index[0].reshape(NW, EPW // KC1, KC1)
    dst1 = edge_index[1].reshape(NW, EPW // KC1, KC1)

    part1, cnt1 = _make_seg_sum1(KC1)(
        x, src1, dst1, jnp.zeros((RPT, DIN), f32), jnp.zeros((RPT, DCNT), f32),
        jnp.ones((KC1, DCNT), f32))

    bf = batch.astype(f32).reshape(NB, 1, BN)
    wl2p = jnp.pad(W_l2, ((0, 0), (0, DQ - DOUT)))
    p, cinv16, ph1 = _tc1(part1, cnt1, x, bf, W_l1, b_l1.reshape(1, DH),
                          W_r1, wl2p)

    srcq = edge_index[0].reshape(NW, EPW // KCQ, KCQ)
    dstq = edge_index[1].reshape(NW, EPW // KCQ, KCQ)
    part2 = _make_seg_sum(DQ, KCQ)(p, srcq, dstq, jnp.zeros((RPT, DQ), f32))

    wr2p = jnp.pad(W_r2, ((0, 0), (0, DQ - DOUT)))
    bl2p = jnp.pad(b_l2, (0, DQ - DOUT)).reshape(1, DQ)
    hg, lsm = _tc2(part2, cinv16, bf, ph1, wr2p, bl2p)
    return hg[:, :DOUT], lsm[:, :DOUT]
